# baseline (speedup 1.0000x reference)
import functools

import jax
import jax.numpy as jnp
from jax import lax
from jax.experimental import pallas as pl
from jax.experimental.pallas import tpu as pltpu
from jax.experimental.pallas import tpu_sc as plsc

_V = 100000
_VP = 100096
_D = 50
_DP = 56
_B = 16384
_L = 50
_LP = 56
_OUTB = 3 * _B
_NC = 2
_NS = 16
_CB = 4096
_NH = _B // _CB
_NG = _CB // 128
_NSTEP = _L * _NH


_G = 10
_NPH = _L // _G


def _body(wordT, e0, e1, e2, out,
          idx_spm, col_v, idx_v, stage_a, stage_b,
          sem_oa, sem_ob):
    cid = lax.axis_index("c")
    sid = lax.axis_index("s")
    wid = sid * _NC + cid

    def out_copy(q, t, dblk, di, buf, sem):
        l = q // _NH
        h = lax.rem(q, _NH)
        bblk0 = (t * _B + h * _CB) // 128
        return pltpu.make_async_copy(
            buf, out.at[l, dblk, pl.ds(bblk0, _CB // 128), di, :], sem)

    def gather(base, stage_v):
        @plsc.parallel_loop(0, _NG, unroll=4)
        def g_body(i):
            for j in range(8):
                off = pl.multiple_of(j * 16, 16)
                vals = plsc.load_gather(
                    col_v, [idx_v[pl.ds(base + i * 128 + off, 16)]])
                stage_v[i, pl.ds(off, 16)] = vals

    def process(tbl, t, d, pbase):
        dblk = d // 8
        di = lax.rem(d, 8)
        qbase = _NH * pbase
        pltpu.sync_copy(tbl.at[d, pl.ds(0, _V)], col_v)

        def step(q, base, buf_s, sem_o):
            @pl.when(q >= qbase + 2)
            def _():
                out_copy(q - 2, t, dblk, di, buf_s, sem_o).wait()

            gather(base, buf_s)
            out_copy(q, t, dblk, di, buf_s, sem_o).start()

        def l_iter(m, carry):
            q0 = _NH * (pbase + m)
            pltpu.sync_copy(idx_spm.at[m, pl.ds(0, 2 * _CB)], idx_v)
            step(q0, 0, stage_a, sem_oa)
            step(q0 + 1, _CB, stage_b, sem_ob)
            pltpu.sync_copy(idx_spm.at[m, pl.ds(2 * _CB, 2 * _CB)], idx_v)
            step(q0 + 2, 0, stage_a, sem_oa)
            step(q0 + 3, _CB, stage_b, sem_ob)
            return carry

        lax.fori_loop(0, _G, l_iter, 0)
        qend = _NH * (pbase + _G)
        out_copy(qend - 2, t, dblk, di, stage_a, sem_oa).wait()
        out_copy(qend - 1, t, dblk, di, stage_b, sem_ob).wait()

    def phase(p, carry):
        pbase = p * _G
        plsc.subcore_barrier()

        @pl.when(sid < _G)
        def _():
            pltpu.sync_copy(wordT.at[pl.ds(pbase + sid, 1)],
                            idx_spm.at[pl.ds(sid, 1)])

        plsc.subcore_barrier()

        for t, tbl in enumerate((e0, e1, e2)):
            process(tbl, t, wid, pbase)
            a = (0, 18, 4)[t]
            r = lax.rem(wid - a + 32, 32)

            @pl.when(r < 18)
            def _(tbl=tbl, t=t, r=r):
                process(tbl, t, 32 + r, pbase)
        return carry

    lax.fori_loop(0, _NPH, phase, 0)


_col_gather = pl.kernel(
    _body,
    mesh=plsc.VectorSubcoreMesh(core_axis_name="c", subcore_axis_name="s"),
    out_type=jax.ShapeDtypeStruct((_L, _DP // 8, _OUTB // 128, 8, 128),
                                  jnp.float32),
    compiler_params=pltpu.CompilerParams(
        use_tc_tiling_on_sc=False, needs_layout_passes=False),
    scratch_types=[
        pltpu.VMEM_SHARED((_G, _B), jnp.int32),
        pltpu.VMEM((_V,), jnp.float32),
        pltpu.VMEM((2 * _CB,), jnp.int32),
        pltpu.VMEM((_CB // 128, 128), jnp.float32),
        pltpu.VMEM((_CB // 128, 128), jnp.float32),
        pltpu.SemaphoreType.DMA,
        pltpu.SemaphoreType.DMA,
    ],
)


@jax.jit
def _run(wordT, t0, t1, t2):
    outp = _col_gather(wordT, t0, t1, t2)
    out = outp.transpose(2, 4, 0, 1, 3).reshape(_OUTB, _L, _DP)
    return out[:, :, :_D]


def kernel(word, E0, E1, E2):
    wordT = jnp.pad(word.astype(jnp.int32).T, ((0, _LP - _L), (0, 0)))
    tp = lambda E: jnp.pad(E.T, ((0, _DP - _D), (0, _VP - _V)))
    return _run(wordT, tp(E0), tp(E1), tp(E2))


# speedup vs baseline: 1.2117x; 1.2117x over previous
import functools

import jax
import jax.numpy as jnp
from jax import lax
from jax.experimental import pallas as pl
from jax.experimental.pallas import tpu as pltpu
from jax.experimental.pallas import tpu_sc as plsc

_V = 100000
_VP = 100096
_D = 50
_DP = 56
_B = 16384
_L = 50
_LP = 56
_OUTB = 3 * _B
_NC = 2
_NS = 16
_CB = 4096
_NH = _B // _CB
_NG = _CB // 128
_NSTEP = _L * _NH


def _body(wordT, e0, e1, e2, out,
          col_v, idx_a, idx_b, stage_a, stage_b,
          sem_ia, sem_ib, sem_oa, sem_ob):
    cid = lax.axis_index("c")
    sid = lax.axis_index("s")
    wid = sid * _NC + cid

    def idx_copy(l, half, buf, sem):
        return pltpu.make_async_copy(
            wordT.at[l, pl.ds(half * (2 * _CB), 2 * _CB)], buf, sem)

    def out_copy(q, t, dblk, di, buf, sem):
        l = q // _NH
        h = lax.rem(q, _NH)
        bblk0 = (t * _B + h * _CB) // 128
        return pltpu.make_async_copy(
            buf, out.at[l, dblk, pl.ds(bblk0, _CB // 128), di, :], sem)

    def gather(idx_v, base, stage_v):
        @plsc.parallel_loop(0, _NG, unroll=8)
        def g_body(i):
            for j in range(8):
                off = pl.multiple_of(j * 16, 16)
                vals = plsc.load_gather(
                    col_v, [idx_v[pl.ds(base + i * 128 + off, 16)]])
                stage_v[i, pl.ds(off, 16)] = vals

    def process(tbl, t, d):
        dblk = d // 8
        di = lax.rem(d, 8)
        pltpu.sync_copy(tbl.at[d, pl.ds(0, _V)], col_v)
        idx_copy(0, 0, idx_a, sem_ia).start()
        idx_copy(0, 1, idx_b, sem_ib).start()

        def step(q, buf_i, base, buf_s, sem_o):
            @pl.when(q >= 2)
            def _():
                out_copy(q - 2, t, dblk, di, buf_s, sem_o).wait()

            gather(buf_i, base, buf_s)
            out_copy(q, t, dblk, di, buf_s, sem_o).start()

        def l_iter(m, carry):
            q0 = _NH * m
            idx_copy(m, 0, idx_a, sem_ia).wait()
            step(q0, idx_a, 0, stage_a, sem_oa)
            step(q0 + 1, idx_a, _CB, stage_b, sem_ob)

            @pl.when(m + 1 < _L)
            def _():
                idx_copy(m + 1, 0, idx_a, sem_ia).start()

            idx_copy(m, 1, idx_b, sem_ib).wait()
            step(q0 + 2, idx_b, 0, stage_a, sem_oa)
            step(q0 + 3, idx_b, _CB, stage_b, sem_ob)

            @pl.when(m + 1 < _L)
            def _():
                idx_copy(m + 1, 1, idx_b, sem_ib).start()

            return carry

        lax.fori_loop(0, _L, l_iter, 0)
        out_copy(_NSTEP - 2, t, dblk, di, stage_a, sem_oa).wait()
        out_copy(_NSTEP - 1, t, dblk, di, stage_b, sem_ob).wait()

    for t, tbl in enumerate((e0, e1, e2)):
        process(tbl, t, wid)
        a = (0, 18, 4)[t]
        r = lax.rem(wid - a + 32, 32)

        @pl.when(r < 18)
        def _(tbl=tbl, t=t, r=r):
            process(tbl, t, 32 + r)


_col_gather = pl.kernel(
    _body,
    mesh=plsc.VectorSubcoreMesh(core_axis_name="c", subcore_axis_name="s"),
    out_type=jax.ShapeDtypeStruct((_L, _DP // 8, _OUTB // 128, 8, 128),
                                  jnp.float32),
    compiler_params=pltpu.CompilerParams(
        use_tc_tiling_on_sc=False, needs_layout_passes=False),
    scratch_types=[
        pltpu.VMEM((_V,), jnp.float32),
        pltpu.VMEM((2 * _CB,), jnp.int32),
        pltpu.VMEM((2 * _CB,), jnp.int32),
        pltpu.VMEM((_CB // 128, 128), jnp.float32),
        pltpu.VMEM((_CB // 128, 128), jnp.float32),
        pltpu.SemaphoreType.DMA,
        pltpu.SemaphoreType.DMA,
        pltpu.SemaphoreType.DMA,
        pltpu.SemaphoreType.DMA,
    ],
)


@jax.jit
def _run(wordT, t0, t1, t2):
    outp = _col_gather(wordT, t0, t1, t2)
    out = outp.transpose(2, 4, 0, 1, 3).reshape(_OUTB, _L, _DP)
    return out[:, :, :_D]


def kernel(word, E0, E1, E2):
    wordT = jnp.pad(word.astype(jnp.int32).T, ((0, _LP - _L), (0, 0)))
    tp = lambda E: jnp.pad(E.T, ((0, _DP - _D), (0, _VP - _V)))
    return _run(wordT, tp(E0), tp(E1), tp(E2))
